# baseline (speedup 1.0000x reference)
import functools

import jax
import jax.numpy as jnp
from jax import lax
from jax.experimental import pallas as pl
from jax.experimental.pallas import tpu as pltpu
from jax.experimental.pallas import tpu_sc as plsc

VOCAB = 1000000
EMBED = 64
MAXLEN = 200
BATCH = 4096
NW = 32
BB = BATCH // NW
NBUF = 5
NLANES = 16
EBLK = EMBED // 8

_mesh = plsc.VectorSubcoreMesh(core_axis_name="c", subcore_axis_name="s")


@functools.partial(
    pl.kernel,
    out_type=jax.ShapeDtypeStruct((MAXLEN, EBLK, NW, 8, BB), jnp.float32),
    mesh=_mesh,
    scratch_types=[
        pltpu.VMEM((MAXLEN, EMBED), jnp.float32),
        pltpu.VMEM((MAXLEN, BB), jnp.int32),
        [pltpu.VMEM((BB, EMBED), jnp.float32) for _ in range(NBUF)],
        [pltpu.VMEM((EBLK, 1, 8, BB + 1), jnp.float32) for _ in range(NBUF)],
        [pltpu.SemaphoreType.DMA for _ in range(NBUF)],
        [pltpu.SemaphoreType.DMA for _ in range(NBUF)],
    ],
    compiler_params=pltpu.CompilerParams(use_tc_tiling_on_sc=False,
                                         needs_layout_passes=False),
)
def _embed_sc(xt_hbm, table_hbm, fix_hbm, out_hbm,
              fix_v, idx_v, rows_v, outb_v, gsem, ssem):
    wid = lax.axis_index("s") * 2 + lax.axis_index("c")
    b0 = wid * BB
    pltpu.sync_copy(fix_hbm, fix_v)
    pltpu.sync_copy(xt_hbm.at[:, pl.ds(b0, BB)], idx_v)

    def gather_start(b, t):
        pltpu.async_copy(table_hbm.at[idx_v.at[t]], rows_v[b], gsem[b])

    def gather_wait(b, t):
        pltpu.make_async_copy(table_hbm.at[idx_v.at[t]], rows_v[b],
                              gsem[b]).wait()

    def out_slice(t):
        return out_hbm.at[t, :, pl.ds(wid, 1)]

    def ob_slice(b):
        return outb_v[b].at[:, :, :, pl.ds(0, BB)]

    def store_start(b, t):
        pltpu.async_copy(ob_slice(b), out_slice(t), ssem[b])

    def store_wait(b, t):
        pltpu.make_async_copy(ob_slice(b), out_slice(t), ssem[b]).wait()

    def add_transpose(b, t):
        rows = rows_v[b]
        ob = outb_v[b]
        lanes = lax.iota(jnp.int32, NLANES)
        zeros = jnp.zeros((NLANES,), jnp.int32)
        e_ids = [lanes + (g * NLANES) for g in range(EMBED // NLANES)]
        eb_ids = [e >> 3 for e in e_ids]
        ei_ids = [e & 7 for e in e_ids]
        frow = [fix_v[t, pl.ds(g * NLANES, NLANES)]
                for g in range(EMBED // NLANES)]

        def b_body(bi, _):
            for u in range(2):
                brow = bi * 2 + u
                bsplat = jnp.full((NLANES,), brow, jnp.int32)
                for g in range(EMBED // NLANES):
                    vals = rows[brow, pl.ds(g * NLANES, NLANES)] + frow[g]
                    plsc.store_scatter(
                        ob, [eb_ids[g], zeros, ei_ids[g], bsplat], vals)
            return ()

        lax.fori_loop(0, BB // 2, b_body, ())

    for b in range(3):
        gather_start(b, b)

    def group_body(g, _):
        for b in range(NBUF):
            t = g * NBUF + b
            gather_wait(b, t)
            add_transpose(b, t)
            store_start(b, t)

            bn = (b + 3) % NBUF

            @pl.when(t + 3 < MAXLEN)
            def _():
                @pl.when(t >= 2)
                def _():
                    store_wait(bn, t - 2)
                gather_start(bn, t + 3)
        return ()

    lax.fori_loop(0, MAXLEN // NBUF, group_body, ())

    for b in range(NBUF):
        store_wait(b, MAXLEN - NBUF + b)


def kernel(x, input_table, fix_embedding):
    out5 = _embed_sc(x.T, input_table, fix_embedding)
    y = jnp.transpose(out5, (2, 4, 0, 1, 3)).reshape(BATCH, MAXLEN, EMBED)
    return y


# speedup vs baseline: 1.4252x; 1.4252x over previous
import functools

import jax
import jax.numpy as jnp
from jax import lax
from jax.experimental import pallas as pl
from jax.experimental.pallas import tpu as pltpu
from jax.experimental.pallas import tpu_sc as plsc

VOCAB = 1000000
EMBED = 64
MAXLEN = 200
BATCH = 4096
NW = 32
BB = BATCH // NW
NBUF = 5
NLANES = 16
EBLK = EMBED // 8

_mesh = plsc.VectorSubcoreMesh(core_axis_name="c", subcore_axis_name="s")


@functools.partial(
    pl.kernel,
    out_type=jax.ShapeDtypeStruct((MAXLEN, EBLK, NW, 8, BB), jnp.float32),
    mesh=_mesh,
    scratch_types=[
        pltpu.VMEM((MAXLEN, EMBED), jnp.float32),
        pltpu.VMEM((MAXLEN, BB), jnp.int32),
        [pltpu.VMEM((BB, EMBED), jnp.float32) for _ in range(NBUF)],
        [pltpu.VMEM((EBLK, 1, 8, BB + 1), jnp.float32) for _ in range(NBUF)],
        [pltpu.SemaphoreType.DMA for _ in range(NBUF)],
        [pltpu.SemaphoreType.DMA for _ in range(NBUF)],
    ],
    compiler_params=pltpu.CompilerParams(use_tc_tiling_on_sc=False,
                                         needs_layout_passes=False),
)
def _embed_sc(xt_hbm, table_hbm, fix_hbm, out_hbm,
              fix_v, idx_v, rows_v, outb_v, gsem, ssem):
    wid = lax.axis_index("s") * 2 + lax.axis_index("c")
    b0 = wid * BB
    pltpu.sync_copy(fix_hbm, fix_v)
    pltpu.sync_copy(xt_hbm.at[:, pl.ds(b0, BB)], idx_v)

    def gather_start(b, t):
        pltpu.async_copy(table_hbm.at[idx_v.at[t]], rows_v[b], gsem[b])

    def gather_wait(b, t):
        pltpu.make_async_copy(table_hbm.at[idx_v.at[t]], rows_v[b],
                              gsem[b]).wait()

    def out_slice(t):
        return out_hbm.at[t, :, pl.ds(wid, 1)]

    def ob_slice(b):
        return outb_v[b].at[:, :, :, pl.ds(0, BB)]

    def store_start(b, t):
        pltpu.async_copy(ob_slice(b), out_slice(t), ssem[b])

    def store_wait(b, t):
        pltpu.make_async_copy(ob_slice(b), out_slice(t), ssem[b]).wait()

    def add_transpose(b, t):
        rows = rows_v[b]
        ob = outb_v[b]
        lanes = lax.iota(jnp.int32, NLANES)
        zeros = jnp.zeros((NLANES,), jnp.int32)
        e_ids = [lanes + (g * NLANES) for g in range(EMBED // NLANES)]
        eb_ids = [e >> 3 for e in e_ids]
        ei_ids = [e & 7 for e in e_ids]
        frow = [fix_v[t, pl.ds(g * NLANES, NLANES)]
                for g in range(EMBED // NLANES)]

        @plsc.parallel_loop(0, BB, step=2, unroll=4)
        def b_body(bi):
            for u in range(2):
                brow = bi + u
                bsplat = jnp.full((NLANES,), brow, jnp.int32)
                for g in range(EMBED // NLANES):
                    vals = rows[brow, pl.ds(g * NLANES, NLANES)] + frow[g]
                    plsc.store_scatter(
                        ob, [eb_ids[g], zeros, ei_ids[g], bsplat], vals)

    for b in range(3):
        gather_start(b, b)

    def group_body(g, _):
        for b in range(NBUF):
            t = g * NBUF + b
            gather_wait(b, t)
            add_transpose(b, t)
            store_start(b, t)

            bn = (b + 3) % NBUF

            @pl.when(t + 3 < MAXLEN)
            def _():
                @pl.when(t >= 2)
                def _():
                    store_wait(bn, t - 2)
                gather_start(bn, t + 3)
        return ()

    lax.fori_loop(0, MAXLEN // NBUF, group_body, ())

    for b in range(NBUF):
        store_wait(b, MAXLEN - NBUF + b)


def kernel(x, input_table, fix_embedding):
    out5 = _embed_sc(x.T, input_table, fix_embedding)
    y = jnp.transpose(out5, (2, 4, 0, 1, 3)).reshape(BATCH, MAXLEN, EMBED)
    return y


# speedup vs baseline: 1.4702x; 1.0315x over previous
import functools

import jax
import jax.numpy as jnp
from jax import lax
from jax.experimental import pallas as pl
from jax.experimental.pallas import tpu as pltpu
from jax.experimental.pallas import tpu_sc as plsc

VOCAB = 1000000
EMBED = 64
MAXLEN = 200
BATCH = 4096
NW = 32
BB = BATCH // NW
NBUF = 4
NLANES = 16
EBLK = EMBED // 8
PADROW = 2 * EMBED

_mesh = plsc.VectorSubcoreMesh(core_axis_name="c", subcore_axis_name="s")


@functools.partial(
    pl.kernel,
    out_type=jax.ShapeDtypeStruct((MAXLEN, EBLK, NW, 8, BB), jnp.float32),
    mesh=_mesh,
    scratch_types=[
        pltpu.VMEM((MAXLEN, EMBED), jnp.float32),
        [pltpu.VMEM((BB,), jnp.int32) for _ in range(NBUF)],
        [pltpu.VMEM((BB, PADROW), jnp.float32) for _ in range(NBUF)],
        [pltpu.VMEM((EBLK, 1, 8, BB + 1), jnp.float32) for _ in range(NBUF)],
        [pltpu.SemaphoreType.DMA for _ in range(NBUF)],
        [pltpu.SemaphoreType.DMA for _ in range(NBUF)],
        [pltpu.SemaphoreType.DMA for _ in range(NBUF)],
    ],
    compiler_params=pltpu.CompilerParams(use_tc_tiling_on_sc=False,
                                         needs_layout_passes=False),
)
def _embed_sc(xt_hbm, table_hbm, fix_hbm, out_hbm,
              fix_v, idx_v, rows_v, outb_v, isem, gsem, ssem):
    wid = lax.axis_index("s") * 2 + lax.axis_index("c")
    b0 = wid * BB
    pltpu.sync_copy(fix_hbm, fix_v)

    def idx_start(b, t):
        pltpu.async_copy(xt_hbm.at[t, pl.ds(b0, BB)], idx_v[b], isem[b])

    def idx_wait(b, t):
        pltpu.make_async_copy(xt_hbm.at[t, pl.ds(b0, BB)], idx_v[b],
                              isem[b]).wait()

    def gather_start(b):
        pltpu.async_copy(table_hbm.at[idx_v[b]], rows_v[b], gsem[b])

    def gather_wait(b):
        pltpu.make_async_copy(table_hbm.at[idx_v[b]], rows_v[b],
                              gsem[b]).wait()

    def out_slice(t):
        return out_hbm.at[t, :, pl.ds(wid, 1)]

    def ob_slice(b):
        return outb_v[b].at[:, :, :, pl.ds(0, BB)]

    def store_start(b, t):
        pltpu.async_copy(ob_slice(b), out_slice(t), ssem[b])

    def store_wait(b, t):
        pltpu.make_async_copy(ob_slice(b), out_slice(t), ssem[b]).wait()

    def add_transpose(b, t):
        rows = rows_v[b]
        ob = outb_v[b]
        lanes = lax.iota(jnp.int32, NLANES)
        zeros = jnp.zeros((NLANES,), jnp.int32)
        e_ids = [lanes + (g * NLANES) for g in range(EMBED // NLANES)]
        eb_ids = [e >> 3 for e in e_ids]
        ei_ids = [e & 7 for e in e_ids]
        frow = [fix_v[t, pl.ds(g * NLANES, NLANES)]
                for g in range(EMBED // NLANES)]

        @plsc.parallel_loop(0, BB, step=2, unroll=4)
        def b_body(bi):
            for u in range(2):
                brow = bi + u
                bsplat = jnp.full((NLANES,), brow, jnp.int32)
                for g in range(EMBED // NLANES):
                    vals = rows[brow, pl.ds(g * NLANES, NLANES)] + frow[g]
                    plsc.store_scatter(
                        ob, [eb_ids[g], zeros, ei_ids[g], bsplat], vals)

    for b in range(3):
        idx_start(b, b)
    for b in range(2):
        idx_wait(b, b)
        gather_start(b)

    def group_body(g, _):
        for b in range(NBUF):
            t = g * NBUF + b
            bi = (b + 3) % NBUF

            @pl.when(t + 3 < MAXLEN)
            def _():
                idx_start(bi, t + 3)

            gather_wait(b)
            add_transpose(b, t)
            store_start(b, t)

            bn = (b + 2) % NBUF

            @pl.when(t + 2 < MAXLEN)
            def _():
                @pl.when(t >= 2)
                def _():
                    store_wait(bn, t - 2)
                idx_wait(bn, t + 2)
                gather_start(bn)
        return ()

    lax.fori_loop(0, MAXLEN // NBUF, group_body, ())

    for b in range(NBUF):
        store_wait(b, MAXLEN - NBUF + b)


_TC_COLS = 2048
_TC_NB = -(-VOCAB // _TC_COLS)


def _transpose_body(tab_ref, out_ref):
    blk = tab_ref[...]
    eye = jnp.eye(EMBED, dtype=jnp.float32)
    out_ref[:, 0:EMBED] = jax.lax.dot_general(
        blk, eye, (((0,), (0,)), ((), ())),
        preferred_element_type=jnp.float32)


def _transpose_tc(tab_t):
    return pl.pallas_call(
        _transpose_body,
        grid=(_TC_NB,),
        in_specs=[pl.BlockSpec((EMBED, _TC_COLS), lambda i: (0, i))],
        out_specs=pl.BlockSpec((_TC_COLS, PADROW), lambda i: (i, 0)),
        out_shape=jax.ShapeDtypeStruct((VOCAB, PADROW), jnp.float32),
    )(tab_t)


def kernel(x, input_table, fix_embedding):
    table_pad = _transpose_tc(input_table.T)
    out5 = _embed_sc(x.T, table_pad, fix_embedding)
    y = jnp.transpose(out5, (2, 4, 0, 1, 3)).reshape(BATCH, MAXLEN, EMBED)
    return y
